# baseline (device time: 60072 ns/iter reference)
import jax
import jax.numpy as jnp
from jax import lax
from jax.experimental import pallas as pl
from jax.experimental.pallas import tpu as pltpu

BM = 1024
EPS = 1e-5


def kernel(x, gamma):
    m, n = x.shape
    n_global = 2 * n
    nblocks = m // BM
    sub, lane = BM // 128, 128
    gamma2d = gamma.reshape(1, n)

    def body(x_ref, g_ref, out_ref, xsave, send_buf, recv_buf,
             send_sem, recv_sem):
        g = pl.program_id(0)
        my_x = lax.axis_index("x")
        my_y = lax.axis_index("y")
        nbr = (my_x, 1 - my_y)

        rdma = pltpu.make_async_remote_copy(
            src_ref=send_buf,
            dst_ref=recv_buf,
            send_sem=send_sem,
            recv_sem=recv_sem,
            device_id=nbr,
            device_id_type=pl.DeviceIdType.MESH,
        )

        @pl.when(g == 0)
        def _():
            barrier_sem = pltpu.get_barrier_semaphore()
            pl.semaphore_signal(
                barrier_sem, inc=1,
                device_id=nbr, device_id_type=pl.DeviceIdType.MESH,
            )
            pl.semaphore_wait(barrier_sem, 1)

        @pl.when(g < nblocks)
        def _():
            xb = x_ref[...]
            xsave[g] = xb.astype(jnp.bfloat16)
            partial = jnp.sum(xb * xb, axis=1)
            send_buf[g] = partial.reshape(sub, lane)

            @pl.when(g == nblocks - 1)
            def _():
                rdma.start()

        @pl.when(g >= nblocks)
        def _():
            @pl.when(g == nblocks)
            def _():
                rdma.wait_recv()

            h = g - nblocks
            total = send_buf[h] + recv_buf[h]
            inv_rms = lax.rsqrt(total / n_global + EPS)
            xb3 = xsave[h].astype(jnp.float32).reshape(sub, lane, n)
            out3 = xb3 * g_ref[...].reshape(1, 1, n) * inv_rms.reshape(sub, lane, 1)
            out_ref[...] = out3.reshape(BM, n).astype(out_ref.dtype)

            @pl.when(g == 2 * nblocks - 1)
            def _():
                rdma.wait_send()

    return pl.pallas_call(
        body,
        grid=(2 * nblocks,),
        out_shape=jax.ShapeDtypeStruct((m, n), jnp.bfloat16),
        in_specs=[
            pl.BlockSpec((BM, n), lambda g: (jnp.minimum(g, nblocks - 1), 0)),
            pl.BlockSpec((1, n), lambda g: (0, 0)),
        ],
        out_specs=pl.BlockSpec(
            (BM, n), lambda g: (jnp.maximum(g - nblocks, 0), 0)
        ),
        scratch_shapes=[
            pltpu.VMEM((nblocks, BM, n), jnp.bfloat16),
            pltpu.VMEM((nblocks, sub, lane), jnp.float32),
            pltpu.VMEM((nblocks, sub, lane), jnp.float32),
            pltpu.SemaphoreType.DMA,
            pltpu.SemaphoreType.DMA,
        ],
        compiler_params=pltpu.CompilerParams(
            collective_id=0,
            dimension_semantics=("arbitrary",),
            vmem_limit_bytes=64 * 1024 * 1024,
        ),
    )(x, gamma2d)
